# baseline (speedup 1.0000x reference)
import functools

import jax
import jax.numpy as jnp
from jax import lax
from jax.experimental import pallas as pl
from jax.experimental.pallas import tpu as pltpu
from jax.experimental.pallas import tpu_sc as plsc

B = 16384
DIM = 64
NW = 32
ROWS_PER_W = B // NW
GROUPS = ROWS_PER_W // 16


def _body(m_hbm, g_hbm, midx_hbm, gidx_hbm, wb_hbm, out_hbm,
          midx_v, gidx_v, m_v, g_v, res_v, wb_v, sem):
    wid = lax.axis_index("s") * 2 + lax.axis_index("c")
    base = wid * ROWS_PER_W

    pltpu.sync_copy(midx_hbm.at[pl.ds(base, ROWS_PER_W)], midx_v)
    pltpu.sync_copy(gidx_hbm.at[pl.ds(base, ROWS_PER_W)], gidx_v)
    pltpu.sync_copy(wb_hbm, wb_v)

    cm = pltpu.async_copy(m_hbm.at[midx_v], m_v, sem)
    cg = pltpu.async_copy(g_hbm.at[gidx_v], g_v, sem)
    cm.wait()
    cg.wait()

    w = wb_v[0, :]
    bb = wb_v[1, :]

    def group(gi, _):
        rows = gi * 16 + lax.iota(jnp.int32, 16)
        zero = jnp.zeros((16,), jnp.float32)
        dot = zero
        nm2 = zero
        ng2 = zero
        for j in range(DIM):
            colv = jnp.full((16,), j, jnp.int32)
            mj = plsc.load_gather(m_v, [rows, colv])
            gj = plsc.load_gather(g_v, [rows, colv])
            dot = dot + mj * gj
            nm2 = nm2 + mj * mj
            ng2 = ng2 + gj * gj
        d = jnp.maximum(nm2, 1e-16) * jnp.maximum(ng2, 1e-16)
        di = plsc.bitcast(d, jnp.int32)
        y = plsc.bitcast(jnp.int32(0x5F3759DF) - (di >> 1), jnp.float32)
        for _ in range(3):
            y = y * (1.5 - 0.5 * d * y * y)
        cos = dot * y
        z = cos * w + bb
        sig = 1.0 / (1.0 + jnp.exp(-z))
        res_v[pl.ds(gi * 16, 16)] = sig
        return 0

    lax.fori_loop(0, GROUPS, group, 0)

    pltpu.sync_copy(res_v, out_hbm.at[pl.ds(base, ROWS_PER_W)])


@jax.jit
def _run(m_table, g_table, midx, gidx, wb):
    mesh = plsc.VectorSubcoreMesh(core_axis_name="c", subcore_axis_name="s")
    f = functools.partial(
        pl.kernel,
        mesh=mesh,
        out_type=jax.ShapeDtypeStruct((B,), jnp.float32),
        scratch_types=[
            pltpu.VMEM((ROWS_PER_W,), jnp.int32),
            pltpu.VMEM((ROWS_PER_W,), jnp.int32),
            pltpu.VMEM((ROWS_PER_W, DIM), jnp.float32),
            pltpu.VMEM((ROWS_PER_W, DIM), jnp.float32),
            pltpu.VMEM((ROWS_PER_W,), jnp.float32),
            pltpu.VMEM((2, 16), jnp.float32),
            pltpu.SemaphoreType.DMA,
        ],
        compiler_params=pltpu.CompilerParams(
            needs_layout_passes=False, use_tc_tiling_on_sc=False
        ),
    )(_body)
    return f(m_table, g_table, midx, gidx, wb)


def kernel(x, m_table, g_table, fc_w, fc_b):
    midx = x[:, 0].astype(jnp.int32)
    gidx = x[:, 1].astype(jnp.int32)
    wb = jnp.stack([
        jnp.broadcast_to(fc_w.reshape(()), (16,)),
        jnp.broadcast_to(fc_b.reshape(()), (16,)),
    ]).astype(jnp.float32)
    out = _run(m_table, g_table, midx, gidx, wb)
    return out.reshape(B, 1)
